# baseline (speedup 1.0000x reference)
import functools

import jax
import jax.numpy as jnp
from jax import lax
from jax.experimental import pallas as pl
from jax.experimental.pallas import tpu as pltpu


def _ca_kernel(x_ref, w_ref, o_ref, sh_ref, sw_ref, *, H, W, Cr):
    f32 = jnp.float32
    B = x_ref.shape[0]
    C = x_ref.shape[3]
    HW = H * W
    Hh = H // 2
    h = pl.program_id(1)

    @pl.when(h == 0)
    def _compute_gates():
        w1 = w_ref[:Cr]
        wht = w_ref[Cr:2 * Cr]
        wwt = w_ref[2 * Cr:]

        j = lax.broadcasted_iota(jnp.int32, (H + W, HW), 0)
        p = lax.broadcasted_iota(jnp.int32, (H + W, HW), 1)
        pt = jnp.where(j < H,
                       jnp.where(p // W == j, f32(1.0 / W), f32(0.0)),
                       jnp.where(p % W == j - H, f32(1.0 / H), f32(0.0)))

        x2 = x_ref[...].reshape(B * HW, C)
        for b in range(B):
            xb = x2[b * HW:(b + 1) * HW]
            pooled = jnp.dot(pt, xb, preferred_element_type=f32)
            z = jnp.maximum(
                lax.dot_general(pooled, w1,
                                (((1,), (1,)), ((), ())),
                                preferred_element_type=f32), 0.0)
            sh_ref[b] = jax.nn.sigmoid(
                jnp.dot(z[:H], wht, preferred_element_type=f32))
            sw_ref[b] = jax.nn.sigmoid(
                jnp.dot(z[H:], wwt, preferred_element_type=f32))

    off = h * Hh
    for b in range(B):
        xh = x_ref[b, pl.ds(off, Hh)]
        o_ref[b] = (xh * sh_ref[b, pl.ds(off, Hh)][:, None, :]
                    * sw_ref[b][None, :, :])


def kernel(x, w1, wh, ww):
    N, C, H, W = x.shape
    HW = H * W
    Cr = w1.shape[0]

    B = 8
    while N % B:
        B //= 2

    xt = jnp.transpose(x, (0, 2, 3, 1))
    wall = jnp.concatenate([w1, wh.T, ww.T], axis=0)

    body = functools.partial(_ca_kernel, H=H, W=W, Cr=Cr)

    flops = int(N * (2 * HW * (H + W) * C
                     + 2 * (H + W) * C * Cr * 3
                     + 2 * HW * C))
    bytes_acc = int(2 * N * C * HW * 4 + 3 * C * Cr * 4)

    out_t = pl.pallas_call(
        body,
        out_shape=jax.ShapeDtypeStruct((N, H, W, C), x.dtype),
        grid=(N // B, 2),
        in_specs=[
            pl.BlockSpec((B, H, W, C), lambda n, h: (n, 0, 0, 0)),
            pl.BlockSpec((3 * Cr, C), lambda n, h: (0, 0)),
        ],
        out_specs=pl.BlockSpec((B, H // 2, W, C), lambda n, h: (n, h, 0, 0)),
        scratch_shapes=[pltpu.VMEM((B, H, C), jnp.float32),
                        pltpu.VMEM((B, W, C), jnp.float32)],
        compiler_params=pltpu.CompilerParams(
            dimension_semantics=("parallel", "arbitrary"),
            vmem_limit_bytes=48 << 20),
        cost_estimate=pl.CostEstimate(
            flops=flops,
            transcendentals=int(N * C * (H + W)),
            bytes_accessed=bytes_acc),
    )(xt, wall)

    return jnp.transpose(out_t, (0, 3, 1, 2))


# speedup vs baseline: 1.4686x; 1.4686x over previous
import functools

import jax
import jax.numpy as jnp
from jax import lax
from jax.experimental import pallas as pl
from jax.experimental.pallas import tpu as pltpu


def _ca_kernel(x_ref, w_ref, o_ref, *, H, W, Cr):
    f32 = jnp.float32
    B = x_ref.shape[0]
    C = x_ref.shape[3]
    HW = H * W

    w1 = w_ref[:Cr]
    wht = w_ref[Cr:2 * Cr]
    wwt = w_ref[2 * Cr:]

    j = lax.broadcasted_iota(jnp.int32, (H + W, HW), 0)
    p = lax.broadcasted_iota(jnp.int32, (H + W, HW), 1)
    pt = jnp.where(j < H,
                   jnp.where(p // W == j, f32(1.0 / W), f32(0.0)),
                   jnp.where(p % W == j - H, f32(1.0 / H), f32(0.0)))

    x4 = x_ref[...]
    x2 = x4.reshape(B * HW, C)

    for b in range(B):
        xb = x2[b * HW:(b + 1) * HW]
        pooled = jnp.dot(pt, xb, preferred_element_type=f32)
        z = jnp.maximum(
            lax.dot_general(pooled, w1,
                            (((1,), (1,)), ((), ())),
                            preferred_element_type=f32), 0.0)
        sh = jax.nn.sigmoid(
            jnp.dot(z[:H], wht, preferred_element_type=f32))
        sw = jax.nn.sigmoid(
            jnp.dot(z[H:], wwt, preferred_element_type=f32))
        o_ref[b] = x4[b] * sh[:, None, :] * sw[None, :, :]


def kernel(x, w1, wh, ww):
    N, C, H, W = x.shape
    HW = H * W
    Cr = w1.shape[0]

    B = 8
    while N % B:
        B //= 2

    xt = jnp.transpose(x, (0, 2, 3, 1))
    wall = jnp.concatenate([w1, wh.T, ww.T], axis=0)

    body = functools.partial(_ca_kernel, H=H, W=W, Cr=Cr)

    flops = int(N * (2 * HW * (H + W) * C
                     + 2 * (H + W) * C * Cr * 3
                     + 2 * HW * C))
    bytes_acc = int(2 * N * C * HW * 4 + 3 * C * Cr * 4)

    out_t = pl.pallas_call(
        body,
        out_shape=jax.ShapeDtypeStruct((N, H, W, C), x.dtype),
        grid=(N // B,),
        in_specs=[
            pl.BlockSpec((B, H, W, C), lambda n: (n, 0, 0, 0)),
            pl.BlockSpec((3 * Cr, C), lambda n: (0, 0)),
        ],
        out_specs=pl.BlockSpec((B, H, W, C), lambda n: (n, 0, 0, 0)),
        compiler_params=pltpu.CompilerParams(
            dimension_semantics=("parallel",),
            vmem_limit_bytes=48 << 20),
        cost_estimate=pl.CostEstimate(
            flops=flops,
            transcendentals=int(N * C * (H + W)),
            bytes_accessed=bytes_acc),
    )(xt, wall)

    return jnp.transpose(out_t, (0, 3, 1, 2))


# speedup vs baseline: 1.4695x; 1.0006x over previous
import functools

import jax
import jax.numpy as jnp
from jax import lax
from jax.experimental import pallas as pl
from jax.experimental.pallas import tpu as pltpu


def _ca_kernel(x_ref, w_ref, o_ref, *, H, W, Cr):
    f32 = jnp.float32
    B = x_ref.shape[0]
    C = x_ref.shape[3]
    HW = H * W

    w1 = w_ref[:Cr]
    wht = w_ref[Cr:2 * Cr]
    wwt = w_ref[2 * Cr:]

    j = lax.broadcasted_iota(jnp.int32, (H + W, HW), 0)
    p = lax.broadcasted_iota(jnp.int32, (H + W, HW), 1)
    pt = jnp.where(j < H,
                   jnp.where(p // W == j, f32(1.0 / W), f32(0.0)),
                   jnp.where(p % W == j - H, f32(1.0 / H), f32(0.0)))

    x4 = x_ref[...]
    x2 = x4.reshape(B * HW, C)

    for b in range(B):
        xb = x2[b * HW:(b + 1) * HW]
        pooled = jnp.dot(pt, xb, preferred_element_type=f32)
        z = jnp.maximum(
            lax.dot_general(pooled, w1,
                            (((1,), (1,)), ((), ())),
                            preferred_element_type=f32), 0.0)
        sh = jax.nn.sigmoid(
            jnp.dot(z[:H], wht, preferred_element_type=f32))
        sw = jax.nn.sigmoid(
            jnp.dot(z[H:], wwt, preferred_element_type=f32))
        o_ref[b] = x4[b] * sh[:, None, :] * sw[None, :, :]


def kernel(x, w1, wh, ww):
    N, C, H, W = x.shape
    HW = H * W
    Cr = w1.shape[0]

    B = 8
    while N % B:
        B //= 2

    xt = jnp.transpose(x, (0, 2, 3, 1))
    wall = jnp.concatenate([w1, wh.T, ww.T], axis=0)

    body = functools.partial(_ca_kernel, H=H, W=W, Cr=Cr)

    flops = int(N * (2 * HW * (H + W) * C
                     + 2 * (H + W) * C * Cr * 3
                     + 2 * HW * C))
    bytes_acc = int(2 * N * C * HW * 4 + 3 * C * Cr * 4)

    out_t = pl.pallas_call(
        body,
        out_shape=jax.ShapeDtypeStruct((N, H, W, C), x.dtype),
        grid=(N // B,),
        in_specs=[
            pl.BlockSpec((B, H, W, C), lambda n: (n, 0, 0, 0)),
            pl.BlockSpec((3 * Cr, C), lambda n: (0, 0)),
        ],
        out_specs=pl.BlockSpec((B, H, W, C), lambda n: (n, 0, 0, 0)),
        compiler_params=pltpu.CompilerParams(
            dimension_semantics=("arbitrary",),
            vmem_limit_bytes=48 << 20),
        cost_estimate=pl.CostEstimate(
            flops=flops,
            transcendentals=int(N * C * (H + W)),
            bytes_accessed=bytes_acc),
    )(xt, wall)

    return jnp.transpose(out_t, (0, 3, 1, 2))
